# Initial kernel scaffold; baseline (speedup 1.0000x reference)
import jax
import jax.numpy as jnp
from jax.experimental import pallas as pl


def kernel(edge_feat, W0, W1, W2):
    raise NotImplementedError("write your pallas kernel here")


# speedup vs baseline: 2.9801x; 2.9801x over previous
import functools

import jax
import jax.numpy as jnp
from jax import lax
from jax.experimental import pallas as pl
from jax.experimental.pallas import tpu as pltpu
from jax.experimental.pallas import tpu_sc as plsc

DIM = 128
L = 16
NC, NS = 2, 16
NW = NC * NS
CHUNK = 80


def _sc_kernel_body(R0, R1, R2, BPW, NCHUNK,
                    ef_hbm, w0_hbm, w1_hbm, w2_hbm,
                    out_hbm, tbl_hbm,
                    ef_v, w0_v, w1_v, w2_v, t_v, combo_v, rows_v,
                    gsem, ssem):
    NT = R0 * R1 * R2
    NTP = (NT + 7) // 8 * 8
    wid = lax.axis_index("s") * NC + lax.axis_index("c")
    base = wid * BPW

    pltpu.sync_copy(w0_hbm, w0_v)
    pltpu.sync_copy(w1_hbm, w1_v)
    pltpu.sync_copy(w2_hbm, w2_v)
    pltpu.sync_copy(ef_hbm.at[pl.ds(base, BPW)], ef_v)

    def build_row(r, _):
        a = jnp.minimum(r // (R1 * R2), R0 - 1)
        rem = r % (R1 * R2)
        b = jnp.minimum(rem // R2, R1 - 1)
        c = rem % R2
        for k in range(DIM // L):
            sl = pl.ds(k * L, L)
            t_v[r, sl] = w0_v[a, sl] + w1_v[b, sl] + w2_v[c, sl]
        return _

    lax.fori_loop(0, NTP, build_row, 0)

    pltpu.sync_copy(t_v, tbl_hbm.at[pl.ds(wid * NTP, NTP)])

    lanes = lax.iota(jnp.int32, L)
    col0 = jnp.zeros((L,), jnp.int32)
    col1 = jnp.ones((L,), jnp.int32)
    col2 = jnp.full((L,), 2, jnp.int32)
    tbl_base = wid * NTP

    def chunk(t, _):
        off = t * CHUNK
        for g in range(CHUNK // L):
            rows16 = off + g * L + lanes
            i0 = plsc.load_gather(ef_v, [rows16, col0])
            i1 = plsc.load_gather(ef_v, [rows16, col1])
            i2 = plsc.load_gather(ef_v, [rows16, col2])
            combo_v[pl.ds(g * L, L)] = (
                i0 * (R1 * R2) + i1 * R2 + i2 + tbl_base)
        pltpu.async_copy(tbl_hbm.at[combo_v], rows_v, gsem).wait()
        pltpu.async_copy(rows_v, out_hbm.at[pl.ds(base + off, CHUNK)],
                         ssem).wait()
        return _

    lax.fori_loop(0, NCHUNK, chunk, 0)


def kernel(edge_feat, W0, W1, W2):
    E = edge_feat.shape[0]
    R0, R1, R2 = W0.shape[0], W1.shape[0], W2.shape[0]
    NTP = (R0 * R1 * R2 + 7) // 8 * 8
    assert E % (NW * CHUNK) == 0
    BPW = E // NW
    NCHUNK = BPW // CHUNK

    mesh = plsc.VectorSubcoreMesh(core_axis_name="c", subcore_axis_name="s")
    f = pl.kernel(
        functools.partial(_sc_kernel_body, R0, R1, R2, BPW, NCHUNK),
        out_type=(
            jax.ShapeDtypeStruct((E, DIM), jnp.float32),
            jax.ShapeDtypeStruct((NW * NTP, DIM), jnp.float32),
        ),
        mesh=mesh,
        compiler_params=pltpu.CompilerParams(
            needs_layout_passes=False, use_tc_tiling_on_sc=False),
        scratch_types=[
            pltpu.VMEM((BPW, 3), jnp.int32),
            pltpu.VMEM((R0, DIM), jnp.float32),
            pltpu.VMEM((R1, DIM), jnp.float32),
            pltpu.VMEM((R2, DIM), jnp.float32),
            pltpu.VMEM((NTP, DIM), jnp.float32),
            pltpu.VMEM((CHUNK,), jnp.int32),
            pltpu.VMEM((CHUNK, DIM), jnp.float32),
            pltpu.SemaphoreType.DMA,
            pltpu.SemaphoreType.DMA,
        ],
    )
    out, _ = f(edge_feat, W0, W1, W2)
    return out
